# baseline (speedup 1.0000x reference)
import jax
import jax.numpy as jnp
from jax import lax
from jax.experimental import pallas as pl
from jax.experimental.pallas import tpu as pltpu
from jax.experimental.pallas import tpu_sc as plsc

_GATHER_DNUMS = lax.GatherDimensionNumbers(
    offset_dims=(), collapsed_slice_dims=(0,), start_index_map=(0,))


def _permute(v, idx):
    return lax.gather(v, idx.reshape(16, 1), _GATHER_DNUMS, (1,),
                      mode=lax.GatherScatterMode.PROMISE_IN_BOUNDS)


_B, _L, _D = 4096, 200, 64
_V = 1000000
_VP = 1000064
_PR = _VP // 2
_NCH = _V // 128
_NW = 32
_RPW = _B // _NW
_NBUF = 4
_SPLIT = 104


def _transpose_body(tabt_hbm, tail_hbm, scr_hbm, inb, outb, skew, isems,
                    osems):
    wid = lax.axis_index("s") * 2 + lax.axis_index("c")
    lanes = lax.broadcasted_iota(jnp.int32, (16,), 0)

    rot_idx = [(lanes + dp) & 15 for dp in range(16)]
    col_idx = [lanes * 16 + ((vp - lanes) & 15) for vp in range(16)]

    start = wid * 244

    def fire_in(c, b):
        off = pl.multiple_of(c * 128, 128)
        pltpu.async_copy(tabt_hbm.at[:, pl.ds(off, 128)], inb.at[b],
                         isems[b])

    def wait_in(b):
        pltpu.make_async_copy(tabt_hbm.at[:, pl.ds(0, 128)], inb.at[b],
                              isems[b]).wait()

    def fire_out(c, b):
        off = pl.multiple_of(c * 64, 8)
        pltpu.async_copy(outb.at[b], scr_hbm.at[pl.ds(off, 64)], osems[b])

    def wait_out(b):
        pltpu.make_async_copy(outb.at[b], scr_hbm.at[pl.ds(0, 64)],
                              osems[b]).wait()

    def transpose_slab(c, b):
        def vblock(vblk, carry):
            v0 = vblk * 16
            for dblk in range(4):
                for dp in range(16):
                    row = inb[b, dblk * 16 + dp, pl.ds(v0, 16)]
                    skew[pl.ds(dblk * 256 + dp * 16, 16)] = \
                        _permute(row, rot_idx[dp])
            for dblk in range(4):
                for vp in range(16):
                    vals = plsc.load_gather(skew, [col_idx[vp] + dblk * 256])
                    v = v0 + vp
                    outb[b, (v >> 1), pl.ds((v & 1) * _D + dblk * 16, 16)] \
                        = vals
            return carry

        lax.fori_loop(0, 8, vblock, 0)
        fire_out(c, b)

    fire_in(start, 0)

    def step(j, carry):
        c0 = start + 2 * j

        fire_in(c0 + 1, 1)
        wait_in(0)

        @pl.when(j >= 1)
        def _():
            wait_out(0)

        transpose_slab(c0, 0)

        @pl.when(j < 121)
        def _():
            fire_in(c0 + 2, 0)

        wait_in(1)

        @pl.when(j >= 1)
        def _():
            wait_out(1)

        transpose_slab(c0 + 1, 1)
        return carry

    lax.fori_loop(0, 122, step, 0)
    wait_out(0)
    wait_out(1)

    @pl.when(wid < 4)
    def _():
        c = 7808 + wid
        fire_in(c, 0)
        wait_in(0)
        transpose_slab(c, 0)
        wait_out(0)

    @pl.when(wid == 0)
    def _():
        pltpu.sync_copy(tail_hbm, scr_hbm.at[pl.ds(_V // 128 * 64, 32)])


_transposer = pl.kernel(
    _transpose_body,
    out_type=jax.ShapeDtypeStruct((_PR, 128), jnp.float32),
    mesh=plsc.VectorSubcoreMesh(core_axis_name="c", subcore_axis_name="s"),
    scratch_types=[
        pltpu.VMEM((2, _D, 128), jnp.float32),
        pltpu.VMEM((2, _D, 128), jnp.float32),
        pltpu.VMEM((1024,), jnp.float32),
        [pltpu.SemaphoreType.DMA, pltpu.SemaphoreType.DMA],
        [pltpu.SemaphoreType.DMA, pltpu.SemaphoreType.DMA],
    ],
    compiler_params=pltpu.CompilerParams(
        needs_layout_passes=False, use_tc_tiling_on_sc=True),
)


def _encode_body(tok_hbm, table_hbm, out_hbm, idx_v, bufs, obuf, row0_v,
                 sem0, sem1, sem2, sem3):
    sems = (sem0, sem1, sem2, sem3)
    wid = lax.axis_index("s") * 2 + lax.axis_index("c")
    base = wid * _RPW

    pltpu.sync_copy(tok_hbm.at[pl.ds(base * _L, _RPW * _L)], idx_v)
    pltpu.sync_copy(table_hbm.at[pl.ds(0, 1)], row0_v)
    row0 = [row0_v[0, pl.ds(d * 16, 16)] for d in range(4)]
    lanes = lax.broadcasted_iota(jnp.int32, (16,), 0)

    def fire(r, b):
        buf = bufs.at[b]
        pltpu.async_copy(table_hbm.at[idx_v.at[pl.ds(r * _L, _SPLIT)]],
                         buf.at[pl.ds(0, _SPLIT)], sems[b])
        pltpu.async_copy(
            table_hbm.at[idx_v.at[pl.ds(r * _L + _SPLIT, _L - _SPLIT)]],
            buf.at[pl.ds(_SPLIT, _L - _SPLIT)], sems[b])

    def process(r, b, prefetch_r):
        buf = bufs.at[b]
        pltpu.make_async_copy(table_hbm.at[pl.ds(0, _L)], buf, sems[b]).wait()

        nz = plsc.all_reduce_population_count(
            idx_v[pl.ds(r * _L, 16)] == 0)
        for k in range(1, 12):
            nz = nz + plsc.all_reduce_population_count(
                idx_v[pl.ds(r * _L + k * 16, 16)] == 0)
        tail = (idx_v[pl.ds(r * _L + _L - 16, 16)] == 0) & (lanes >= 8)
        nz = nz + plsc.all_reduce_population_count(tail)

        zero = jnp.zeros((16,), jnp.float32)

        def acc_body(j, accs):
            accs = list(accs)
            rr = j * 8
            for u in range(8):
                for d in range(4):
                    slot = d * 2 + (u & 1)
                    accs[slot] = accs[slot] + buf[rr + u, pl.ds(d * 16, 16)]
            return tuple(accs)

        accs = lax.fori_loop(0, _L // 8, acc_body, (zero,) * 8)

        if prefetch_r is not None:
            fire(prefetch_r, b)

        nzf = nz.astype(jnp.float32)
        inv = 1.0 / (_L - nz).astype(jnp.float32)
        for d in range(4):
            res = (accs[d * 2] + accs[d * 2 + 1] - nzf * row0[d]) * inv
            obuf[pl.ds(r * _D + d * 16, 16)] = res

    for b in range(_NBUF):
        fire(b, b)

    def outer(k, carry):
        for b in range(_NBUF):
            r = k * _NBUF + b
            process(r, b, r + _NBUF)
        return carry

    lax.fori_loop(0, _RPW // _NBUF - 1, outer, 0)
    for b in range(_NBUF):
        process(_RPW - _NBUF + b, b, None)

    pltpu.sync_copy(obuf, out_hbm.at[pl.ds(base * _D, _RPW * _D)])


_encoder = pl.kernel(
    _encode_body,
    out_type=jax.ShapeDtypeStruct((_B * _D,), jnp.float32),
    mesh=plsc.VectorSubcoreMesh(core_axis_name="c", subcore_axis_name="s"),
    scratch_types=[
        pltpu.VMEM((_RPW * _L,), jnp.int32),
        pltpu.VMEM((_NBUF, _L, _D), jnp.float32),
        pltpu.VMEM((_RPW * _D,), jnp.float32),
        pltpu.VMEM((1, _D), jnp.float32),
        pltpu.SemaphoreType.DMA,
        pltpu.SemaphoreType.DMA,
        pltpu.SemaphoreType.DMA,
        pltpu.SemaphoreType.DMA,
    ],
    compiler_params=pltpu.CompilerParams(
        needs_layout_passes=False, use_tc_tiling_on_sc=False),
)


@jax.jit
def kernel(token_indices, aligned_embeddings):
    table_t = aligned_embeddings.T
    tail = aligned_embeddings[_NCH * 128:].reshape(32, 128)
    scratch = _transposer(table_t, tail)
    out = _encoder(token_indices.reshape(-1),
                   scratch.reshape(_VP, _D))
    return out.reshape(_B, _D)


# speedup vs baseline: 1.9620x; 1.9620x over previous
import jax
import jax.numpy as jnp
from jax import lax
from jax.experimental import pallas as pl
from jax.experimental.pallas import tpu as pltpu
from jax.experimental.pallas import tpu_sc as plsc

_GATHER_DNUMS = lax.GatherDimensionNumbers(
    offset_dims=(), collapsed_slice_dims=(0,), start_index_map=(0,))


def _permute(v, idx):
    return lax.gather(v, idx.reshape(16, 1), _GATHER_DNUMS, (1,),
                      mode=lax.GatherScatterMode.PROMISE_IN_BOUNDS)


_B, _L, _D = 4096, 200, 64
_V = 1000000
_VP = 1000064
_PR = _VP // 2
_NCH = _V // 128
_NW = 32
_RPW = _B // _NW
_NBUF = 4
_SPLIT = 104


def _transpose_body(tabt_hbm, tail_hbm, scr_hbm, inb, outb, skew, isems,
                    osems):
    wid = lax.axis_index("s") * 2 + lax.axis_index("c")
    lanes = lax.broadcasted_iota(jnp.int32, (16,), 0)

    rot_idx = [(lanes + dp) & 15 for dp in range(16)]
    col_idx = [lanes * 16 + ((vp - lanes) & 15) for vp in range(16)]

    start = wid * 244

    def fire_in(c, b):
        off = pl.multiple_of(c * 128, 128)
        pltpu.async_copy(tabt_hbm.at[:, pl.ds(off, 128)], inb.at[b],
                         isems[b])

    def wait_in(b):
        pltpu.make_async_copy(tabt_hbm.at[:, pl.ds(0, 128)], inb.at[b],
                              isems[b]).wait()

    def fire_out(c, b):
        off = pl.multiple_of(c * 64, 8)
        pltpu.async_copy(outb.at[b], scr_hbm.at[pl.ds(off, 64)], osems[b])

    def wait_out(b):
        pltpu.make_async_copy(outb.at[b], scr_hbm.at[pl.ds(0, 64)],
                              osems[b]).wait()

    def transpose_slab(c, b):
        def vblock(vblk):
            v0 = vblk * 16
            sk = vblk * 1024
            for dblk in range(4):
                for dp in range(16):
                    row = inb[b, dblk * 16 + dp, pl.ds(v0, 16)]
                    skew[pl.ds(sk + dblk * 256 + dp * 16, 16)] = \
                        _permute(row, rot_idx[dp])
            for dblk in range(4):
                for vp in range(16):
                    vals = plsc.load_gather(
                        skew, [col_idx[vp] + (sk + dblk * 256)])
                    v = v0 + vp
                    outb[b, (v >> 1), pl.ds((v & 1) * _D + dblk * 16, 16)] \
                        = vals

        plsc.parallel_loop(0, 8, unroll=2)(vblock)
        fire_out(c, b)

    fire_in(start, 0)

    def step(j, carry):
        c0 = start + 2 * j

        fire_in(c0 + 1, 1)
        wait_in(0)

        @pl.when(j >= 1)
        def _():
            wait_out(0)

        transpose_slab(c0, 0)

        @pl.when(j < 121)
        def _():
            fire_in(c0 + 2, 0)

        wait_in(1)

        @pl.when(j >= 1)
        def _():
            wait_out(1)

        transpose_slab(c0 + 1, 1)
        return carry

    lax.fori_loop(0, 122, step, 0)
    wait_out(0)
    wait_out(1)

    @pl.when(wid < 4)
    def _():
        c = 7808 + wid
        fire_in(c, 0)
        wait_in(0)
        transpose_slab(c, 0)
        wait_out(0)

    @pl.when(wid == 0)
    def _():
        pltpu.sync_copy(tail_hbm, scr_hbm.at[pl.ds(_V // 128 * 64, 32)])


_transposer = pl.kernel(
    _transpose_body,
    out_type=jax.ShapeDtypeStruct((_PR, 128), jnp.float32),
    mesh=plsc.VectorSubcoreMesh(core_axis_name="c", subcore_axis_name="s"),
    scratch_types=[
        pltpu.VMEM((2, _D, 128), jnp.float32),
        pltpu.VMEM((2, _D, 128), jnp.float32),
        pltpu.VMEM((8192,), jnp.float32),
        [pltpu.SemaphoreType.DMA, pltpu.SemaphoreType.DMA],
        [pltpu.SemaphoreType.DMA, pltpu.SemaphoreType.DMA],
    ],
    compiler_params=pltpu.CompilerParams(
        needs_layout_passes=False, use_tc_tiling_on_sc=True),
)


def _encode_body(tok_hbm, table_hbm, out_hbm, idx_v, bufs, obuf, row0_v,
                 sem0, sem1, sem2, sem3):
    sems = (sem0, sem1, sem2, sem3)
    wid = lax.axis_index("s") * 2 + lax.axis_index("c")
    base = wid * _RPW

    pltpu.sync_copy(tok_hbm.at[pl.ds(base * _L, _RPW * _L)], idx_v)
    pltpu.sync_copy(table_hbm.at[pl.ds(0, 1)], row0_v)
    row0 = [row0_v[0, pl.ds(d * 16, 16)] for d in range(4)]
    lanes = lax.broadcasted_iota(jnp.int32, (16,), 0)

    def fire(r, b):
        buf = bufs.at[b]
        pltpu.async_copy(table_hbm.at[idx_v.at[pl.ds(r * _L, _SPLIT)]],
                         buf.at[pl.ds(0, _SPLIT)], sems[b])
        pltpu.async_copy(
            table_hbm.at[idx_v.at[pl.ds(r * _L + _SPLIT, _L - _SPLIT)]],
            buf.at[pl.ds(_SPLIT, _L - _SPLIT)], sems[b])

    def process(r, b, prefetch_r):
        buf = bufs.at[b]
        pltpu.make_async_copy(table_hbm.at[pl.ds(0, _L)], buf, sems[b]).wait()

        nz = plsc.all_reduce_population_count(
            idx_v[pl.ds(r * _L, 16)] == 0)
        for k in range(1, 12):
            nz = nz + plsc.all_reduce_population_count(
                idx_v[pl.ds(r * _L + k * 16, 16)] == 0)
        tail = (idx_v[pl.ds(r * _L + _L - 16, 16)] == 0) & (lanes >= 8)
        nz = nz + plsc.all_reduce_population_count(tail)

        zero = jnp.zeros((16,), jnp.float32)

        def acc_body(j, accs):
            accs = list(accs)
            rr = j * 8
            for u in range(8):
                for d in range(4):
                    slot = d * 2 + (u & 1)
                    accs[slot] = accs[slot] + buf[rr + u, pl.ds(d * 16, 16)]
            return tuple(accs)

        accs = lax.fori_loop(0, _L // 8, acc_body, (zero,) * 8)

        if prefetch_r is not None:
            fire(prefetch_r, b)

        nzf = nz.astype(jnp.float32)
        inv = 1.0 / (_L - nz).astype(jnp.float32)
        for d in range(4):
            res = (accs[d * 2] + accs[d * 2 + 1] - nzf * row0[d]) * inv
            obuf[pl.ds(r * _D + d * 16, 16)] = res

    for b in range(_NBUF):
        fire(b, b)

    def outer(k, carry):
        for b in range(_NBUF):
            r = k * _NBUF + b
            process(r, b, r + _NBUF)
        return carry

    lax.fori_loop(0, _RPW // _NBUF - 1, outer, 0)
    for b in range(_NBUF):
        process(_RPW - _NBUF + b, b, None)

    pltpu.sync_copy(obuf, out_hbm.at[pl.ds(base * _D, _RPW * _D)])


_encoder = pl.kernel(
    _encode_body,
    out_type=jax.ShapeDtypeStruct((_B * _D,), jnp.float32),
    mesh=plsc.VectorSubcoreMesh(core_axis_name="c", subcore_axis_name="s"),
    scratch_types=[
        pltpu.VMEM((_RPW * _L,), jnp.int32),
        pltpu.VMEM((_NBUF, _L, _D), jnp.float32),
        pltpu.VMEM((_RPW * _D,), jnp.float32),
        pltpu.VMEM((1, _D), jnp.float32),
        pltpu.SemaphoreType.DMA,
        pltpu.SemaphoreType.DMA,
        pltpu.SemaphoreType.DMA,
        pltpu.SemaphoreType.DMA,
    ],
    compiler_params=pltpu.CompilerParams(
        needs_layout_passes=False, use_tc_tiling_on_sc=False),
)


@jax.jit
def kernel(token_indices, aligned_embeddings):
    table_t = aligned_embeddings.T
    tail = aligned_embeddings[_NCH * 128:].reshape(32, 128)
    scratch = _transposer(table_t, tail)
    out = _encoder(token_indices.reshape(-1),
                   scratch.reshape(_VP, _D))
    return out.reshape(_B, _D)
